# baseline (speedup 1.0000x reference)
import functools

import jax
import jax.numpy as jnp
from jax import lax
from jax.experimental import pallas as pl
from jax.experimental.pallas import tpu as pltpu
from jax.experimental.pallas import tpu_sc as plsc

B = 1024
P = 128
O = 128
G = 128
K = 4
NW = G + K
H = 2.0 / G

NC = 2
NS = 16
BSC = 256
BTC = B - BSC
NPC = 16
NBC = 2
PB = P // NPC
BB = BSC // NBC
LANES = 16
NBUF = 4
WSTRIDE = 136
PCHUNK = 16


def _basis_math(x):
    i = jnp.clip(3 + jnp.floor((x + 1.0) * (1.0 / H)).astype(jnp.int32), 3, 130)

    def tval(m):
        return jnp.clip(-1.0 + (m.astype(jnp.float32) - 3.0) * H, -1.0, 1.0)

    basis = [jnp.ones_like(x)]
    left, right = [], []
    for j in range(1, K):
        left.append(x - tval(i + (1 - j)))
        right.append(tval(i + j) - x)
        saved = jnp.zeros_like(x)
        new_basis = []
        for r in range(j):
            temp = basis[r] / (right[r] + left[j - 1 - r])
            new_basis.append(saved + right[r] * temp)
            saved = left[j - 1 - r] * temp
        new_basis.append(saved)
        basis = new_basis
    return i, basis


def _basis_body(xt_ref, r0_ref, n_ref):
    i, basis = _basis_math(xt_ref[...])
    r0_ref[...] = i - 3
    for j in range(K):
        n_ref[j] = basis[j]


def _basis(xt):
    return pl.pallas_call(
        _basis_body,
        out_shape=(
            jax.ShapeDtypeStruct((P, BSC), jnp.int32),
            jax.ShapeDtypeStruct((K, P, BSC), jnp.float32),
        ),
    )(xt)


def _tc_spline_body(xt_ref, wf_ref, yt_ref, at_scr):
    pstep = pl.program_id(0)

    @pl.when(pstep == 0)
    def _():
        yt_ref[...] = jnp.zeros_like(yt_ref)

    xblk = xt_ref[...]
    i, basis = _basis_math(xblk)
    g = jax.lax.broadcasted_iota(jnp.int32, (WSTRIDE, BTC), 0)
    for pp in range(PCHUNK):
        d = g - (i[pp:pp + 1, :] - 3)
        a = jnp.where(d == 0, basis[0][pp:pp + 1, :], 0.0)
        a = jnp.where(d == 1, basis[1][pp:pp + 1, :], a)
        a = jnp.where(d == 2, basis[2][pp:pp + 1, :], a)
        a = jnp.where(d == 3, basis[3][pp:pp + 1, :], a)
        at_scr[pp * WSTRIDE:(pp + 1) * WSTRIDE, :] = a
    yt_ref[...] += jax.lax.dot_general(
        wf_ref[...], at_scr[...],
        (((0,), (0,)), ((), ())),
        preferred_element_type=jnp.float32,
    )


def _tc_spline(xtc_t, w_flat):
    return pl.pallas_call(
        _tc_spline_body,
        grid=(P // PCHUNK,),
        in_specs=[
            pl.BlockSpec((PCHUNK, BTC), lambda j: (j, 0)),
            pl.BlockSpec((PCHUNK * WSTRIDE, O), lambda j: (j, 0)),
        ],
        out_specs=pl.BlockSpec((O, BTC), lambda j: (0, 0)),
        out_shape=jax.ShapeDtypeStruct((O, BTC), jnp.float32),
        scratch_shapes=[pltpu.VMEM((PCHUNK * WSTRIDE, BTC), jnp.float32)],
    )(xtc_t, w_flat)


def _sc_body(wp_hbm, r0_hbm, n_hbm, out_hbm, wbuf, r0v, nv, acc, wsem):
    wid = lax.axis_index("c") * NS + lax.axis_index("s")
    pc = lax.div(wid, NBC)
    bc = lax.rem(wid, NBC)
    p0 = pc * PB
    b0 = bc * BB

    pltpu.sync_copy(r0_hbm.at[pl.ds(p0, PB), pl.ds(b0, BB)],
                    r0v.at[:, pl.ds(0, BB)])
    for j in range(K):
        pltpu.sync_copy(n_hbm.at[j, pl.ds(p0, PB), pl.ds(b0, BB)],
                        nv.at[j, :, pl.ds(0, BB)])

    def zero_body(bb, carry):
        for c8 in range(O // LANES):
            acc[bb, pl.ds(c8 * LANES, LANES)] = jnp.zeros((LANES,), jnp.float32)
        return carry

    lax.fori_loop(0, BB, zero_body, 0)

    def wcopy(pp, k):
        return pltpu.make_async_copy(
            wp_hbm.at[p0 + pp],
            wbuf.at[pl.ds(k * WSTRIDE, WSTRIDE)],
            wsem.at[k],
        )

    for pp in range(NBUF - 1):
        wcopy(pp, pp).start()

    def p_body(pp, carry):
        k = lax.rem(pp, NBUF)

        @pl.when(pp + (NBUF - 1) < PB)
        def _():
            nxt = pp + (NBUF - 1)
            wcopy(nxt, lax.rem(nxt, NBUF)).start()

        wcopy(pp, k).wait()
        base_k = k * WSTRIDE

        @plsc.parallel_loop(0, BB, 1)
        def b_body(bb):
            slb = pl.ds(bb, LANES)
            r = base_k + r0v[pp, slb][0]
            n0 = jnp.full((LANES,), nv[0, pp, slb][0], jnp.float32)
            n1 = jnp.full((LANES,), nv[1, pp, slb][0], jnp.float32)
            n2 = jnp.full((LANES,), nv[2, pp, slb][0], jnp.float32)
            n3 = jnp.full((LANES,), nv[3, pp, slb][0], jnp.float32)
            for c8 in range(O // LANES):
                sl = pl.ds(c8 * LANES, LANES)
                v = (wbuf[r, sl] * n0 + wbuf[r + 1, sl] * n1
                     + wbuf[r + 2, sl] * n2 + wbuf[r + 3, sl] * n3)
                plsc.addupdate(acc.at[bb, sl], v)
        return carry

    lax.fori_loop(0, PB, p_body, 0)
    pltpu.sync_copy(acc, out_hbm.at[pc, pl.ds(b0, BB), :])


@functools.lru_cache(maxsize=None)
def _get_sc_spline():
    mesh = plsc.VectorSubcoreMesh(core_axis_name="c", subcore_axis_name="s")
    return pl.kernel(
        _sc_body,
        out_type=jax.ShapeDtypeStruct((NPC, BSC, O), jnp.float32),
        mesh=mesh,
        compiler_params=pltpu.CompilerParams(needs_layout_passes=False),
        scratch_types=[
            pltpu.VMEM((NBUF * WSTRIDE, O), jnp.float32),
            pltpu.VMEM((PB, BB + LANES), jnp.int32),
            pltpu.VMEM((K, PB, BB + LANES), jnp.float32),
            pltpu.VMEM((BB, O), jnp.float32),
            pltpu.SemaphoreType.DMA((NBUF,)),
        ],
    )


def _final_body(x_ref, w131_ref, part_ref, ytc_ref, y_ref):
    x = x_ref[...]
    sx = x * (1.0 / (1.0 + jnp.exp(-x)))
    ysilu = jnp.dot(
        sx, w131_ref[...],
        precision=jax.lax.Precision.HIGHEST,
        preferred_element_type=jnp.float32,
    )
    y = part_ref[0] + part_ref[1]
    for j in range(2, NPC):
        y = y + part_ref[j]
    y_ref[0:BSC, :] = y + ysilu[0:BSC, :]
    y_ref[BSC:B, :] = ytc_ref[...] + ysilu[BSC:B, :]


def _final(x, w131, part, y_tc):
    return pl.pallas_call(
        _final_body,
        out_shape=jax.ShapeDtypeStruct((B, O), jnp.float32),
    )(x, w131, part, y_tc)


def kernel(x, w, t):
    del t
    x_sc = x[:BSC]
    xt = x_sc.T
    xtc_t = x[BSC:].T
    w_perm = jnp.pad(jnp.transpose(w, (1, 0, 2)),
                     ((0, 0), (0, WSTRIDE - NW), (0, 0)))
    w_flat = w_perm.reshape(P * WSTRIDE, O)
    w131 = w[NW - 1]
    r0, n = _basis(xt)
    part = _get_sc_spline()(w_perm, r0, n)
    y_tc = _tc_spline(xtc_t, w_flat).T
    return _final(x, w131, part, y_tc)


# speedup vs baseline: 1.0520x; 1.0520x over previous
import functools

import jax
import jax.numpy as jnp
from jax import lax
from jax.experimental import pallas as pl
from jax.experimental.pallas import tpu as pltpu
from jax.experimental.pallas import tpu_sc as plsc

B = 1024
P = 128
O = 128
G = 128
K = 4
NW = G + K
H = 2.0 / G

NC = 2
NS = 16
BSC = 128
BTC = B - BSC
NPC = 32
NBC = 1
PB = P // NPC
BB = BSC // NBC
RB = max(PB, 8)
LANES = 16
NBUF = 4
WSTRIDE = 136
PCHUNK = 16


def _basis_math(x):
    i = jnp.clip(3 + jnp.floor((x + 1.0) * (1.0 / H)).astype(jnp.int32), 3, 130)

    def tval(m):
        return jnp.clip(-1.0 + (m.astype(jnp.float32) - 3.0) * H, -1.0, 1.0)

    basis = [jnp.ones_like(x)]
    left, right = [], []
    for j in range(1, K):
        left.append(x - tval(i + (1 - j)))
        right.append(tval(i + j) - x)
        saved = jnp.zeros_like(x)
        new_basis = []
        for r in range(j):
            temp = basis[r] / (right[r] + left[j - 1 - r])
            new_basis.append(saved + right[r] * temp)
            saved = left[j - 1 - r] * temp
        new_basis.append(saved)
        basis = new_basis
    return i, basis


def _basis_body(xt_ref, r0_ref, n_ref):
    i, basis = _basis_math(xt_ref[...])
    r0_ref[...] = i - 3
    for j in range(K):
        n_ref[j] = basis[j]


def _basis(xt):
    return pl.pallas_call(
        _basis_body,
        out_shape=(
            jax.ShapeDtypeStruct((P, BSC), jnp.int32),
            jax.ShapeDtypeStruct((K, P, BSC), jnp.float32),
        ),
    )(xt)


def _tc_spline_body(xt_ref, wf_ref, yt_ref, at_scr):
    pstep = pl.program_id(0)

    @pl.when(pstep == 0)
    def _():
        yt_ref[...] = jnp.zeros_like(yt_ref)

    xblk = xt_ref[...]
    i, basis = _basis_math(xblk)
    g = jax.lax.broadcasted_iota(jnp.int32, (WSTRIDE, BTC), 0)
    for pp in range(PCHUNK):
        d = g - (i[pp:pp + 1, :] - 3)
        a = jnp.where(d == 0, basis[0][pp:pp + 1, :], 0.0)
        a = jnp.where(d == 1, basis[1][pp:pp + 1, :], a)
        a = jnp.where(d == 2, basis[2][pp:pp + 1, :], a)
        a = jnp.where(d == 3, basis[3][pp:pp + 1, :], a)
        at_scr[pp * WSTRIDE:(pp + 1) * WSTRIDE, :] = a
    yt_ref[...] += jax.lax.dot_general(
        wf_ref[...], at_scr[...],
        (((0,), (0,)), ((), ())),
        preferred_element_type=jnp.float32,
    )


def _tc_spline(xtc_t, w_flat):
    return pl.pallas_call(
        _tc_spline_body,
        grid=(P // PCHUNK,),
        in_specs=[
            pl.BlockSpec((PCHUNK, BTC), lambda j: (j, 0)),
            pl.BlockSpec((PCHUNK * WSTRIDE, O), lambda j: (j, 0)),
        ],
        out_specs=pl.BlockSpec((O, BTC), lambda j: (0, 0)),
        out_shape=jax.ShapeDtypeStruct((O, BTC), jnp.float32),
        scratch_shapes=[pltpu.VMEM((PCHUNK * WSTRIDE, BTC), jnp.float32)],
    )(xtc_t, w_flat)


def _sc_body(wp_hbm, r0_hbm, n_hbm, out_hbm, wbuf, r0v, nv, acc, wsem):
    wid = lax.axis_index("c") * NS + lax.axis_index("s")
    pc = lax.div(wid, NBC)
    bc = lax.rem(wid, NBC)
    p0 = pc * PB
    b0 = bc * BB

    pal = lax.div(p0, RB) * RB
    off = p0 - pal
    pltpu.sync_copy(r0_hbm.at[pl.ds(pal, RB), pl.ds(b0, BB)],
                    r0v.at[:, pl.ds(0, BB)])
    for j in range(K):
        pltpu.sync_copy(n_hbm.at[j, pl.ds(pal, RB), pl.ds(b0, BB)],
                        nv.at[j, :, pl.ds(0, BB)])

    def zero_body(bb, carry):
        for c8 in range(O // LANES):
            acc[bb, pl.ds(c8 * LANES, LANES)] = jnp.zeros((LANES,), jnp.float32)
        return carry

    lax.fori_loop(0, BB, zero_body, 0)

    def wcopy(pp, k):
        return pltpu.make_async_copy(
            wp_hbm.at[p0 + pp],
            wbuf.at[pl.ds(k * WSTRIDE, WSTRIDE)],
            wsem.at[k],
        )

    for pp in range(NBUF - 1):
        wcopy(pp, pp).start()

    def p_body(pp, carry):
        k = lax.rem(pp, NBUF)

        @pl.when(pp + (NBUF - 1) < PB)
        def _():
            nxt = pp + (NBUF - 1)
            wcopy(nxt, lax.rem(nxt, NBUF)).start()

        wcopy(pp, k).wait()
        base_k = k * WSTRIDE

        ppo = off + pp

        @plsc.parallel_loop(0, BB, 1)
        def b_body(bb):
            slb = pl.ds(bb, LANES)
            r = base_k + r0v[ppo, slb][0]
            n0 = jnp.full((LANES,), nv[0, ppo, slb][0], jnp.float32)
            n1 = jnp.full((LANES,), nv[1, ppo, slb][0], jnp.float32)
            n2 = jnp.full((LANES,), nv[2, ppo, slb][0], jnp.float32)
            n3 = jnp.full((LANES,), nv[3, ppo, slb][0], jnp.float32)
            for c8 in range(O // LANES):
                sl = pl.ds(c8 * LANES, LANES)
                v = (wbuf[r, sl] * n0 + wbuf[r + 1, sl] * n1
                     + wbuf[r + 2, sl] * n2 + wbuf[r + 3, sl] * n3)
                plsc.addupdate(acc.at[bb, sl], v)
        return carry

    lax.fori_loop(0, PB, p_body, 0)
    pltpu.sync_copy(acc, out_hbm.at[pc, pl.ds(b0, BB), :])


@functools.lru_cache(maxsize=None)
def _get_sc_spline():
    mesh = plsc.VectorSubcoreMesh(core_axis_name="c", subcore_axis_name="s")
    return pl.kernel(
        _sc_body,
        out_type=jax.ShapeDtypeStruct((NPC, BSC, O), jnp.float32),
        mesh=mesh,
        compiler_params=pltpu.CompilerParams(needs_layout_passes=False),
        scratch_types=[
            pltpu.VMEM((NBUF * WSTRIDE, O), jnp.float32),
            pltpu.VMEM((RB, BB + LANES), jnp.int32),
            pltpu.VMEM((K, RB, BB + LANES), jnp.float32),
            pltpu.VMEM((BB, O), jnp.float32),
            pltpu.SemaphoreType.DMA((NBUF,)),
        ],
    )


def _final_body(x_ref, w131_ref, part_ref, ytc_ref, y_ref):
    x = x_ref[...]
    sx = x * (1.0 / (1.0 + jnp.exp(-x)))
    ysilu = jnp.dot(
        sx, w131_ref[...],
        precision=jax.lax.Precision.HIGHEST,
        preferred_element_type=jnp.float32,
    )
    y = part_ref[0] + part_ref[1]
    for j in range(2, NPC):
        y = y + part_ref[j]
    y_ref[0:BSC, :] = y + ysilu[0:BSC, :]
    y_ref[BSC:B, :] = ytc_ref[...] + ysilu[BSC:B, :]


def _final(x, w131, part, y_tc):
    return pl.pallas_call(
        _final_body,
        out_shape=jax.ShapeDtypeStruct((B, O), jnp.float32),
    )(x, w131, part, y_tc)


def kernel(x, w, t):
    del t
    x_sc = x[:BSC]
    xt = x_sc.T
    xtc_t = x[BSC:].T
    w_perm = jnp.pad(jnp.transpose(w, (1, 0, 2)),
                     ((0, 0), (0, WSTRIDE - NW), (0, 0)))
    w_flat = w_perm.reshape(P * WSTRIDE, O)
    w131 = w[NW - 1]
    r0, n = _basis(xt)
    part = _get_sc_spline()(w_perm, r0, n)
    y_tc = _tc_spline(xtc_t, w_flat).T
    return _final(x, w131, part, y_tc)


# speedup vs baseline: 1.0609x; 1.0086x over previous
import functools

import jax
import jax.numpy as jnp
from jax import lax
from jax.experimental import pallas as pl
from jax.experimental.pallas import tpu as pltpu
from jax.experimental.pallas import tpu_sc as plsc

B = 1024
P = 128
O = 128
G = 128
K = 4
NW = G + K
H = 2.0 / G

NC = 2
NS = 16
BSC = 128
BTC = B - BSC
NPC = 32
NBC = 1
PB = P // NPC
BB = BSC // NBC
RB = max(PB, 8)
LANES = 16
NBUF = 4
WSTRIDE = 136
PCHUNK = 32


def _basis_math(x):
    i = jnp.clip(3 + jnp.floor((x + 1.0) * (1.0 / H)).astype(jnp.int32), 3, 130)

    def tval(m):
        return jnp.clip(-1.0 + (m.astype(jnp.float32) - 3.0) * H, -1.0, 1.0)

    basis = [jnp.ones_like(x)]
    left, right = [], []
    for j in range(1, K):
        left.append(x - tval(i + (1 - j)))
        right.append(tval(i + j) - x)
        saved = jnp.zeros_like(x)
        new_basis = []
        for r in range(j):
            temp = basis[r] / (right[r] + left[j - 1 - r])
            new_basis.append(saved + right[r] * temp)
            saved = left[j - 1 - r] * temp
        new_basis.append(saved)
        basis = new_basis
    return i, basis


def _basis_body(xt_ref, r0_ref, n_ref):
    i, basis = _basis_math(xt_ref[...])
    r0_ref[...] = i - 3
    for j in range(K):
        n_ref[j] = basis[j]


def _basis(xt):
    return pl.pallas_call(
        _basis_body,
        out_shape=(
            jax.ShapeDtypeStruct((P, BSC), jnp.int32),
            jax.ShapeDtypeStruct((K, P, BSC), jnp.float32),
        ),
    )(xt)


def _tc_spline_body(xt_ref, wf_ref, yt_ref, at_scr):
    pstep = pl.program_id(0)

    @pl.when(pstep == 0)
    def _():
        yt_ref[...] = jnp.zeros_like(yt_ref)

    xblk = xt_ref[...]
    i, basis = _basis_math(xblk)
    g = jax.lax.broadcasted_iota(jnp.int32, (WSTRIDE, BTC), 0)
    for pp in range(PCHUNK):
        d = g - (i[pp:pp + 1, :] - 3)
        a = jnp.where(d == 0, basis[0][pp:pp + 1, :], 0.0)
        a = jnp.where(d == 1, basis[1][pp:pp + 1, :], a)
        a = jnp.where(d == 2, basis[2][pp:pp + 1, :], a)
        a = jnp.where(d == 3, basis[3][pp:pp + 1, :], a)
        at_scr[pp * WSTRIDE:(pp + 1) * WSTRIDE, :] = a
    yt_ref[...] += jax.lax.dot_general(
        wf_ref[...], at_scr[...],
        (((0,), (0,)), ((), ())),
        preferred_element_type=jnp.float32,
    )


def _tc_spline(xtc_t, w_flat):
    return pl.pallas_call(
        _tc_spline_body,
        grid=(P // PCHUNK,),
        in_specs=[
            pl.BlockSpec((PCHUNK, BTC), lambda j: (j, 0)),
            pl.BlockSpec((PCHUNK * WSTRIDE, O), lambda j: (j, 0)),
        ],
        out_specs=pl.BlockSpec((O, BTC), lambda j: (0, 0)),
        out_shape=jax.ShapeDtypeStruct((O, BTC), jnp.float32),
        scratch_shapes=[pltpu.VMEM((PCHUNK * WSTRIDE, BTC), jnp.float32)],
    )(xtc_t, w_flat)


def _sc_body(wp_hbm, r0_hbm, n_hbm, out_hbm, wbuf, r0v, nv, acc, wsem):
    wid = lax.axis_index("c") * NS + lax.axis_index("s")
    pc = lax.div(wid, NBC)
    bc = lax.rem(wid, NBC)
    p0 = pc * PB
    b0 = bc * BB

    pal = lax.div(p0, RB) * RB
    off = p0 - pal
    pltpu.sync_copy(r0_hbm.at[pl.ds(pal, RB), pl.ds(b0, BB)],
                    r0v.at[:, pl.ds(0, BB)])
    for j in range(K):
        pltpu.sync_copy(n_hbm.at[j, pl.ds(pal, RB), pl.ds(b0, BB)],
                        nv.at[j, :, pl.ds(0, BB)])

    def zero_body(bb, carry):
        for c8 in range(O // LANES):
            acc[bb, pl.ds(c8 * LANES, LANES)] = jnp.zeros((LANES,), jnp.float32)
        return carry

    lax.fori_loop(0, BB, zero_body, 0)

    def wcopy(pp, k):
        return pltpu.make_async_copy(
            wp_hbm.at[p0 + pp],
            wbuf.at[pl.ds(k * WSTRIDE, WSTRIDE)],
            wsem.at[k],
        )

    for pp in range(NBUF - 1):
        wcopy(pp, pp).start()

    def p_body(pp, carry):
        k = lax.rem(pp, NBUF)

        @pl.when(pp + (NBUF - 1) < PB)
        def _():
            nxt = pp + (NBUF - 1)
            wcopy(nxt, lax.rem(nxt, NBUF)).start()

        wcopy(pp, k).wait()
        base_k = k * WSTRIDE

        ppo = off + pp

        @plsc.parallel_loop(0, BB, 1)
        def b_body(bb):
            slb = pl.ds(bb, LANES)
            r = base_k + r0v[ppo, slb][0]
            n0 = jnp.full((LANES,), nv[0, ppo, slb][0], jnp.float32)
            n1 = jnp.full((LANES,), nv[1, ppo, slb][0], jnp.float32)
            n2 = jnp.full((LANES,), nv[2, ppo, slb][0], jnp.float32)
            n3 = jnp.full((LANES,), nv[3, ppo, slb][0], jnp.float32)
            for c8 in range(O // LANES):
                sl = pl.ds(c8 * LANES, LANES)
                v = (wbuf[r, sl] * n0 + wbuf[r + 1, sl] * n1
                     + wbuf[r + 2, sl] * n2 + wbuf[r + 3, sl] * n3)
                plsc.addupdate(acc.at[bb, sl], v)
        return carry

    lax.fori_loop(0, PB, p_body, 0)
    pltpu.sync_copy(acc, out_hbm.at[pc, pl.ds(b0, BB), :])


@functools.lru_cache(maxsize=None)
def _get_sc_spline():
    mesh = plsc.VectorSubcoreMesh(core_axis_name="c", subcore_axis_name="s")
    return pl.kernel(
        _sc_body,
        out_type=jax.ShapeDtypeStruct((NPC, BSC, O), jnp.float32),
        mesh=mesh,
        compiler_params=pltpu.CompilerParams(needs_layout_passes=False),
        scratch_types=[
            pltpu.VMEM((NBUF * WSTRIDE, O), jnp.float32),
            pltpu.VMEM((RB, BB + LANES), jnp.int32),
            pltpu.VMEM((K, RB, BB + LANES), jnp.float32),
            pltpu.VMEM((BB, O), jnp.float32),
            pltpu.SemaphoreType.DMA((NBUF,)),
        ],
    )


def _final_body(x_ref, w131_ref, part_ref, ytc_ref, y_ref):
    x = x_ref[...]
    sx = x * (1.0 / (1.0 + jnp.exp(-x)))
    ysilu = jnp.dot(
        sx, w131_ref[...],
        precision=jax.lax.Precision.HIGHEST,
        preferred_element_type=jnp.float32,
    )
    y = part_ref[0] + part_ref[1]
    for j in range(2, NPC):
        y = y + part_ref[j]
    y_ref[0:BSC, :] = y + ysilu[0:BSC, :]
    y_ref[BSC:B, :] = ytc_ref[...] + ysilu[BSC:B, :]


def _final(x, w131, part, y_tc):
    return pl.pallas_call(
        _final_body,
        out_shape=jax.ShapeDtypeStruct((B, O), jnp.float32),
    )(x, w131, part, y_tc)


def kernel(x, w, t):
    del t
    x_sc = x[:BSC]
    xt = x_sc.T
    xtc_t = x[BSC:].T
    w_perm = jnp.pad(jnp.transpose(w, (1, 0, 2)),
                     ((0, 0), (0, WSTRIDE - NW), (0, 0)))
    w_flat = w_perm.reshape(P * WSTRIDE, O)
    w131 = w[NW - 1]
    r0, n = _basis(xt)
    part = _get_sc_spline()(w_perm, r0, n)
    y_tc = _tc_spline(xtc_t, w_flat).T
    return _final(x, w131, part, y_tc)


# speedup vs baseline: 1.0963x; 1.0333x over previous
import functools

import jax
import jax.numpy as jnp
from jax import lax
from jax.experimental import pallas as pl
from jax.experimental.pallas import tpu as pltpu
from jax.experimental.pallas import tpu_sc as plsc

B = 1024
P = 128
O = 128
G = 128
K = 4
NW = G + K
H = 2.0 / G

NC = 2
NS = 16
BSC = 128
BTC = B - BSC
NPC = 32
NBC = 1
PB = P // NPC
BB = BSC // NBC
RB = max(PB, 8)
LANES = 16
NBUF = 4
WSTRIDE = 136
PCHUNK = 32


def _basis_math(x):
    i = jnp.clip(3 + jnp.floor((x + 1.0) * (1.0 / H)).astype(jnp.int32), 3, 130)

    def tval(m):
        return jnp.clip(-1.0 + (m.astype(jnp.float32) - 3.0) * H, -1.0, 1.0)

    basis = [jnp.ones_like(x)]
    left, right = [], []
    for j in range(1, K):
        left.append(x - tval(i + (1 - j)))
        right.append(tval(i + j) - x)
        saved = jnp.zeros_like(x)
        new_basis = []
        for r in range(j):
            temp = basis[r] / (right[r] + left[j - 1 - r])
            new_basis.append(saved + right[r] * temp)
            saved = left[j - 1 - r] * temp
        new_basis.append(saved)
        basis = new_basis
    return i, basis


def _basis_body(xt_ref, r0_ref, n_ref):
    i, basis = _basis_math(xt_ref[...][:, 0:BSC])
    r0_ref[...] = i - 3
    for j in range(K):
        n_ref[j] = basis[j]


def _basis(xt):
    return pl.pallas_call(
        _basis_body,
        out_shape=(
            jax.ShapeDtypeStruct((P, BSC), jnp.int32),
            jax.ShapeDtypeStruct((K, P, BSC), jnp.float32),
        ),
    )(xt)


def _tc_spline_body(xt_ref, wf_ref, yt_ref, at_scr):
    pstep = pl.program_id(0)

    @pl.when(pstep == 0)
    def _():
        yt_ref[...] = jnp.zeros_like(yt_ref)

    xblk = xt_ref[...][:, BSC:B]
    i, basis = _basis_math(xblk)
    g = jax.lax.broadcasted_iota(jnp.int32, (WSTRIDE, BTC), 0)
    for pp in range(PCHUNK):
        d = g - (i[pp:pp + 1, :] - 3)
        a = jnp.where(d == 0, basis[0][pp:pp + 1, :], 0.0)
        a = jnp.where(d == 1, basis[1][pp:pp + 1, :], a)
        a = jnp.where(d == 2, basis[2][pp:pp + 1, :], a)
        a = jnp.where(d == 3, basis[3][pp:pp + 1, :], a)
        at_scr[pp * WSTRIDE:(pp + 1) * WSTRIDE, :] = a
    yt_ref[...] += jax.lax.dot_general(
        wf_ref[...], at_scr[...],
        (((0,), (0,)), ((), ())),
        preferred_element_type=jnp.float32,
    )


def _tc_spline(xtc_t, w_flat):
    return pl.pallas_call(
        _tc_spline_body,
        grid=(P // PCHUNK,),
        in_specs=[
            pl.BlockSpec((PCHUNK, B), lambda j: (j, 0)),
            pl.BlockSpec((PCHUNK * WSTRIDE, O), lambda j: (j, 0)),
        ],
        out_specs=pl.BlockSpec((O, BTC), lambda j: (0, 0)),
        out_shape=jax.ShapeDtypeStruct((O, BTC), jnp.float32),
        scratch_shapes=[pltpu.VMEM((PCHUNK * WSTRIDE, BTC), jnp.float32)],
    )(xtc_t, w_flat)


def _sc_body(wp_hbm, r0_hbm, n_hbm, out_hbm, wbuf, r0v, nv, acc, wsem):
    wid = lax.axis_index("c") * NS + lax.axis_index("s")
    pc = lax.div(wid, NBC)
    bc = lax.rem(wid, NBC)
    p0 = pc * PB
    b0 = bc * BB

    pal = lax.div(p0, RB) * RB
    off = p0 - pal
    pltpu.sync_copy(r0_hbm.at[pl.ds(pal, RB), pl.ds(b0, BB)],
                    r0v.at[:, pl.ds(0, BB)])
    for j in range(K):
        pltpu.sync_copy(n_hbm.at[j, pl.ds(pal, RB), pl.ds(b0, BB)],
                        nv.at[j, :, pl.ds(0, BB)])

    def zero_body(bb, carry):
        for c8 in range(O // LANES):
            acc[bb, pl.ds(c8 * LANES, LANES)] = jnp.zeros((LANES,), jnp.float32)
        return carry

    lax.fori_loop(0, BB, zero_body, 0)

    def wcopy(pp, k):
        return pltpu.make_async_copy(
            wp_hbm.at[p0 + pp],
            wbuf.at[pl.ds(k * WSTRIDE, WSTRIDE)],
            wsem.at[k],
        )

    for pp in range(NBUF - 1):
        wcopy(pp, pp).start()

    def p_body(pp, carry):
        k = lax.rem(pp, NBUF)

        @pl.when(pp + (NBUF - 1) < PB)
        def _():
            nxt = pp + (NBUF - 1)
            wcopy(nxt, lax.rem(nxt, NBUF)).start()

        wcopy(pp, k).wait()
        base_k = k * WSTRIDE

        ppo = off + pp

        @plsc.parallel_loop(0, BB, 1)
        def b_body(bb):
            slb = pl.ds(bb, LANES)
            r = base_k + r0v[ppo, slb][0]
            n0 = jnp.full((LANES,), nv[0, ppo, slb][0], jnp.float32)
            n1 = jnp.full((LANES,), nv[1, ppo, slb][0], jnp.float32)
            n2 = jnp.full((LANES,), nv[2, ppo, slb][0], jnp.float32)
            n3 = jnp.full((LANES,), nv[3, ppo, slb][0], jnp.float32)
            for c8 in range(O // LANES):
                sl = pl.ds(c8 * LANES, LANES)
                v = (wbuf[r, sl] * n0 + wbuf[r + 1, sl] * n1
                     + wbuf[r + 2, sl] * n2 + wbuf[r + 3, sl] * n3)
                plsc.addupdate(acc.at[bb, sl], v)
        return carry

    lax.fori_loop(0, PB, p_body, 0)
    pltpu.sync_copy(acc, out_hbm.at[pc, pl.ds(b0, BB), :])


@functools.lru_cache(maxsize=None)
def _get_sc_spline():
    mesh = plsc.VectorSubcoreMesh(core_axis_name="c", subcore_axis_name="s")
    return pl.kernel(
        _sc_body,
        out_type=jax.ShapeDtypeStruct((NPC, BSC, O), jnp.float32),
        mesh=mesh,
        compiler_params=pltpu.CompilerParams(needs_layout_passes=False),
        scratch_types=[
            pltpu.VMEM((NBUF * WSTRIDE, O), jnp.float32),
            pltpu.VMEM((RB, BB + LANES), jnp.int32),
            pltpu.VMEM((K, RB, BB + LANES), jnp.float32),
            pltpu.VMEM((BB, O), jnp.float32),
            pltpu.SemaphoreType.DMA((NBUF,)),
        ],
    )


def _final_body(x_ref, w131_ref, part_ref, ytc_ref, y_ref):
    x = x_ref[...]
    sx = x * (1.0 / (1.0 + jnp.exp(-x)))
    ysilu = jnp.dot(
        sx, w131_ref[...],
        precision=jax.lax.Precision.HIGHEST,
        preferred_element_type=jnp.float32,
    )
    y = part_ref[0] + part_ref[1]
    for j in range(2, NPC):
        y = y + part_ref[j]
    y_ref[0:BSC, :] = y + ysilu[0:BSC, :]
    y_ref[BSC:B, :] = ytc_ref[...] + ysilu[BSC:B, :]


def _final(x, w131, part, y_tc):
    return pl.pallas_call(
        _final_body,
        out_shape=jax.ShapeDtypeStruct((B, O), jnp.float32),
    )(x, w131, part, y_tc)


def kernel(x, w, t):
    del t
    xt = x.T
    w_perm = jnp.pad(jnp.transpose(w, (1, 0, 2)),
                     ((0, 0), (0, WSTRIDE - NW), (0, 0)))
    w_flat = w_perm.reshape(P * WSTRIDE, O)
    w131 = w[NW - 1]
    r0, n = _basis(xt)
    part = _get_sc_spline()(w_perm, r0, n)
    y_tc = _tc_spline(xt, w_flat).T
    return _final(x, w131, part, y_tc)
